# baseline (speedup 1.0000x reference)
import functools

import jax
import jax.numpy as jnp
from jax import lax
from jax.experimental import pallas as pl
from jax.experimental.pallas import tpu as pltpu
from jax.experimental.pallas import tpu_sc as plsc

_N = 10000
_E = 320000
_CH = 128
_NCHUNK = _E // _CH
_NC = 2
_NS = 16
_NW = _NC * _NS
_RPT = _N // _NS

_BR = 1000
_NB = _N // _BR

_RPT8 = 624
_TAIL = _N - _NS * _RPT8
_ZCHUNKS = ((0, 128), (128, 128), (256, 128), (384, 128), (512, 112))


def _zero_rows(buf, nrows, ncols):
    def body(i, carry):
        for j in range(ncols // 16):
            buf[i, pl.ds(j * 16, 16)] = jnp.zeros((16,), jnp.float32)
        return carry
    lax.fori_loop(0, nrows, body, 0)


_CPT = _NCHUNK // _NW
_XTRA = _NCHUNK - _CPT * _NW


@functools.lru_cache(maxsize=None)
def _make_sc_scatter(F):
    stages = 2 if F == 128 else 1
    SR = _CPT // stages
    mesh = plsc.VectorSubcoreMesh(core_axis_name="c", subcore_axis_name="s")

    @functools.partial(
        pl.kernel,
        out_type=jax.ShapeDtypeStruct((_NC, _N, F), jnp.float32),
        mesh=mesh,
        compiler_params=pltpu.CompilerParams(use_tc_tiling_on_sc=False),
        scratch_types=[
            pltpu.VMEM((SR + 1, _CH), jnp.int32),
            pltpu.VMEM((SR + 1, _CH), jnp.int32),
            pltpu.VMEM((2, _CH, F), jnp.float32),
            pltpu.VMEM_SHARED((_N, F), jnp.float32),
            pltpu.SemaphoreType.DMA,
            pltpu.SemaphoreType.DMA,
        ],
    )
    def k(u_hbm, src_hbm, dst_hbm, out_hbm, src_v, dst_v, rows_v, acc_sh,
          sem0, sem1):
        cid = lax.axis_index("c")
        sid = lax.axis_index("s")
        wid = sid * _NC + cid

        _zero_rows(rows_v.at[0], _CH, F)
        for off, sz in _ZCHUNKS:
            pltpu.sync_copy(rows_v.at[0, pl.ds(0, sz)],
                            acc_sh.at[pl.ds(sid * _RPT8 + off, sz)])
        @pl.when(sid == _NS - 1)
        def _():
            pltpu.sync_copy(rows_v.at[0, pl.ds(0, _TAIL)],
                            acc_sh.at[pl.ds(_NS * _RPT8, _TAIL)])

        def load_stage(st):
            pltpu.sync_copy(src_hbm.at[pl.ds(wid * _CPT + st * SR, SR)],
                            src_v.at[pl.ds(0, SR)])
            pltpu.sync_copy(dst_hbm.at[pl.ds(wid * _CPT + st * SR, SR)],
                            dst_v.at[pl.ds(0, SR)])

        load_stage(0)
        @pl.when(wid < _XTRA)
        def _():
            pltpu.sync_copy(src_hbm.at[pl.ds(_NW * _CPT + wid, 1)],
                            src_v.at[pl.ds(SR, 1)])
            pltpu.sync_copy(dst_hbm.at[pl.ds(_NW * _CPT + wid, 1)],
                            dst_v.at[pl.ds(SR, 1)])
        plsc.subcore_barrier()

        sems = (sem0, sem1)

        def start(j, b):
            pltpu.async_copy(u_hbm.at[src_v.at[j]], rows_v.at[b], sems[b])

        def finish(j, b):
            pltpu.make_async_copy(u_hbm.at[src_v.at[j]], rows_v.at[b],
                                  sems[b]).wait()
            pltpu.sync_copy(rows_v.at[b], acc_sh.at[dst_v.at[j]], add=True)

        def pipeline():
            start(0, 0)

            def body(i, carry):
                j0 = 2 * i
                @pl.when(j0 + 1 < SR)
                def _():
                    start(j0 + 1, 1)
                finish(j0, 0)
                @pl.when(j0 + 2 < SR)
                def _():
                    start(j0 + 2, 0)
                @pl.when(j0 + 1 < SR)
                def _():
                    finish(j0 + 1, 1)
                return carry

            lax.fori_loop(0, (SR + 1) // 2, body, 0)

        for st in range(stages):
            if st > 0:
                load_stage(st)
            pipeline()

        @pl.when(wid < _XTRA)
        def _():
            start(SR, 0)
            finish(SR, 0)
        plsc.subcore_barrier()

        pltpu.sync_copy(acc_sh.at[pl.ds(sid * _RPT8, _RPT8)],
                        out_hbm.at[cid, pl.ds(sid * _RPT8, _RPT8)])
        @pl.when(sid == _NS - 1)
        def _():
            pltpu.sync_copy(acc_sh.at[pl.ds(_NS * _RPT8, _TAIL)],
                            out_hbm.at[cid, pl.ds(_NS * _RPT8, _TAIL)])

    return k


@functools.lru_cache(maxsize=None)
def _make_sc_degree():
    F = 16
    mesh = plsc.VectorSubcoreMesh(core_axis_name="c", subcore_axis_name="s")

    @functools.partial(
        pl.kernel,
        out_type=jax.ShapeDtypeStruct((_NC, _N, F), jnp.float32),
        mesh=mesh,
        compiler_params=pltpu.CompilerParams(use_tc_tiling_on_sc=False),
        scratch_types=[
            pltpu.VMEM((_CPT + 1, _CH), jnp.int32),
            pltpu.VMEM((_CH, F), jnp.float32),
            pltpu.VMEM_SHARED((_N, F), jnp.float32),
            pltpu.SemaphoreType.DMA,
        ],
    )
    def k(src_hbm, out_hbm, src_v, ones_v, acc_sh, sem):
        cid = lax.axis_index("c")
        sid = lax.axis_index("s")
        wid = sid * _NC + cid

        _zero_rows(ones_v, _CH, F)
        for off, sz in _ZCHUNKS:
            pltpu.sync_copy(ones_v.at[pl.ds(0, sz)],
                            acc_sh.at[pl.ds(sid * _RPT8 + off, sz)])
        @pl.when(sid == _NS - 1)
        def _():
            pltpu.sync_copy(ones_v.at[pl.ds(0, _TAIL)],
                            acc_sh.at[pl.ds(_NS * _RPT8, _TAIL)])

        pltpu.sync_copy(src_hbm.at[pl.ds(wid * _CPT, _CPT)],
                        src_v.at[pl.ds(0, _CPT)])
        @pl.when(wid < _XTRA)
        def _():
            pltpu.sync_copy(src_hbm.at[pl.ds(_NW * _CPT + wid, 1)],
                            src_v.at[pl.ds(_CPT, 1)])
        plsc.subcore_barrier()

        def fill(i, carry):
            ones_v[i, pl.ds(0, 16)] = jnp.ones((16,), jnp.float32)
            return carry
        lax.fori_loop(0, _CH, fill, 0)

        def body(i, carry):
            pltpu.sync_copy(ones_v, acc_sh.at[src_v.at[i]], add=True)
            return carry

        lax.fori_loop(0, _CPT, body, 0)
        @pl.when(wid < _XTRA)
        def _():
            pltpu.sync_copy(ones_v, acc_sh.at[src_v.at[_CPT]], add=True)
        plsc.subcore_barrier()

        pltpu.sync_copy(acc_sh.at[pl.ds(sid * _RPT8, _RPT8)],
                        out_hbm.at[cid, pl.ds(sid * _RPT8, _RPT8)])
        @pl.when(sid == _NS - 1)
        def _():
            pltpu.sync_copy(acc_sh.at[pl.ds(_NS * _RPT8, _TAIL)],
                            out_hbm.at[cid, pl.ds(_NS * _RPT8, _TAIL)])

    return k


def _rb(shape, idx=None):
    if idx is None:
        idx = lambda i: (i,) + (0,) * (len(shape) - 1)
    return pl.BlockSpec(shape, idx)


@functools.lru_cache(maxsize=None)
def _make_prep1(Fin, Fout):
    def body(degp_ref, x_ref, w_ref, dis_ref, u_ref, acc_ref):
        d = degp_ref[0] + degp_ref[1]
        deg = d[:, 0:1]
        dis = jnp.where(deg > 0.0, lax.rsqrt(jnp.maximum(deg, 1e-30)), 0.0)
        dis_ref[...] = dis
        x = x_ref[...]
        u_ref[...] = x * dis
        acc_ref[...] = jnp.dot(x, w_ref[...],
                               preferred_element_type=jnp.float32)

    return pl.pallas_call(
        body,
        grid=(_NB,),
        in_specs=[
            pl.BlockSpec((_NC, _BR, 16), lambda i: (0, i, 0)),
            _rb((_BR, Fin)),
            pl.BlockSpec((Fin, Fout), lambda i: (0, 0)),
        ],
        out_specs=[
            _rb((_BR, 1)),
            _rb((_BR, Fin)),
            _rb((_BR, Fout)),
        ],
        out_shape=[
            jax.ShapeDtypeStruct((_N, 1), jnp.float32),
            jax.ShapeDtypeStruct((_N, Fin), jnp.float32),
            jax.ShapeDtypeStruct((_N, Fout), jnp.float32),
        ],
    )


@functools.lru_cache(maxsize=None)
def _make_step_u(m, Fin, emit_u):
    def body(*refs):
        if m == 2:
            sp_ref, ta_ref, tb_ref, dis_ref, lam_ref, *outs = refs
        else:
            sp_ref, ta_ref, dis_ref, lam_ref, *outs = refs
            tb_ref = None
        s = 2.0 / lam_ref[0]
        dis = dis_ref[...]
        sm = sp_ref[0] + sp_ref[1]
        new = (-m * s) * (dis * sm) + (m * (s - 1.0)) * ta_ref[...]
        if m == 2:
            new = new - tb_ref[...]
        outs[0][...] = new
        if emit_u:
            outs[1][...] = dis * new

    in_specs = [pl.BlockSpec((_NC, _BR, Fin), lambda i: (0, i, 0)),
                _rb((_BR, Fin))]
    if m == 2:
        in_specs.append(_rb((_BR, Fin)))
    in_specs += [
        _rb((_BR, 1)),
        pl.BlockSpec(memory_space=pltpu.SMEM),
    ]
    out_specs = [_rb((_BR, Fin))]
    out_shape = [jax.ShapeDtypeStruct((_N, Fin), jnp.float32)]
    if emit_u:
        out_specs.append(_rb((_BR, Fin)))
        out_shape.append(jax.ShapeDtypeStruct((_N, Fin), jnp.float32))

    return pl.pallas_call(
        body, grid=(_NB,), in_specs=in_specs,
        out_specs=out_specs, out_shape=out_shape,
    )


@functools.lru_cache(maxsize=None)
def _make_matacc(Fin, Fout):
    def body(new_ref, w_ref, acc_ref, acco_ref):
        acco_ref[...] = acc_ref[...] + jnp.dot(
            new_ref[...], w_ref[...], preferred_element_type=jnp.float32)

    return pl.pallas_call(
        body,
        grid=(_NB,),
        in_specs=[
            _rb((_BR, Fin)),
            pl.BlockSpec((Fin, Fout), lambda i: (0, 0)),
            _rb((_BR, Fout)),
        ],
        out_specs=_rb((_BR, Fout)),
        out_shape=jax.ShapeDtypeStruct((_N, Fout), jnp.float32),
    )


@functools.lru_cache(maxsize=None)
def _make_layer_start(Fp, Fo):
    def body(acc_ref, b_ref, dis_ref, w_ref, h_ref, u_ref, acco_ref):
        h = jax.nn.relu(acc_ref[...] + b_ref[...])
        h_ref[...] = h
        u_ref[...] = h * dis_ref[...]
        acco_ref[...] = jnp.dot(h, w_ref[...],
                                preferred_element_type=jnp.float32)

    return pl.pallas_call(
        body,
        grid=(_NB,),
        in_specs=[
            _rb((_BR, Fp)),
            pl.BlockSpec((1, Fp), lambda i: (0, 0)),
            _rb((_BR, 1)),
            pl.BlockSpec((Fp, Fo), lambda i: (0, 0)),
        ],
        out_specs=[_rb((_BR, Fp)), _rb((_BR, Fp)), _rb((_BR, Fo))],
        out_shape=[
            jax.ShapeDtypeStruct((_N, Fp), jnp.float32),
            jax.ShapeDtypeStruct((_N, Fp), jnp.float32),
            jax.ShapeDtypeStruct((_N, Fo), jnp.float32),
        ],
    )


@functools.lru_cache(maxsize=None)
def _make_pool_mlp(F):
    def body(acc_ref, b_ref, w1_ref, b1_ref, w2_ref, b2_ref, out_ref):
        h = jax.nn.relu(acc_ref[...] + b_ref[...])
        pooled = jnp.sum(h, axis=0, keepdims=True) / jnp.float32(_N)
        t = jax.nn.relu(
            jnp.dot(pooled, w1_ref[...], preferred_element_type=jnp.float32)
            + b1_ref[...])
        out_ref[...] = jnp.dot(
            t, w2_ref[...], preferred_element_type=jnp.float32) + b2_ref[...]

    return pl.pallas_call(
        body,
        out_shape=jax.ShapeDtypeStruct((1, 10), jnp.float32),
    )


def kernel(x, edge_index, lambda_max, batch, W1, b1, W2, b2, W3, b3,
           fc1_w, fc1_b, fc2_w, fc2_b):
    src = edge_index[0].reshape(_NCHUNK, _CH)
    dst = edge_index[1].reshape(_NCHUNK, _CH)
    lam = lambda_max.reshape((1,)).astype(jnp.float32)

    degp = _make_sc_degree()(src)

    dis, u, acc = _make_prep1(128, 32)(degp, x, W1[0])

    def run_hops(u, acc, t0, dis, W, Fin, Fout):
        scat = _make_sc_scatter(Fin)
        matacc = _make_matacc(Fin, Fout)
        sp = scat(u, src, dst)
        t1, u = _make_step_u(1, Fin, True)(sp, t0, dis, lam)
        acc = matacc(t1, W[1], acc)
        for k in range(2, 5):
            sp = scat(u, src, dst)
            emit = k < 4
            outs = _make_step_u(2, Fin, emit)(sp, t1, t0, dis, lam)
            if emit:
                t2, u = outs
            else:
                (t2,) = outs
                u = None
            acc = matacc(t2, W[k], acc)
            t0, t1 = t1, t2
        return acc

    acc = run_hops(u, acc, x, dis, W1, 128, 32)

    h, u, acc2 = _make_layer_start(32, 64)(acc, b1.reshape(1, -1), dis, W2[0])
    acc2 = run_hops(u, acc2, h, dis, W2, 32, 64)

    h, u, acc3 = _make_layer_start(64, 64)(acc2, b2.reshape(1, -1), dis, W3[0])
    acc3 = run_hops(u, acc3, h, dis, W3, 64, 64)

    return _make_pool_mlp(64)(
        acc3, b3.reshape(1, -1), fc1_w, fc1_b.reshape(1, -1),
        fc2_w, fc2_b.reshape(1, -1))


# speedup vs baseline: 1.2227x; 1.2227x over previous
import functools

import jax
import jax.numpy as jnp
from jax import lax
from jax.experimental import pallas as pl
from jax.experimental.pallas import tpu as pltpu
from jax.experimental.pallas import tpu_sc as plsc

_N = 10000
_E = 320000
_CH = 128
_NCHUNK = _E // _CH
_NC = 2
_NS = 16
_NW = _NC * _NS
_RPT = _N // _NS

_BR = 1000
_NB = _N // _BR

_RPT8 = 624
_TAIL = _N - _NS * _RPT8
_ZCHUNKS = ((0, 128), (128, 128), (256, 128), (384, 128), (512, 112))


def _zero_rows(buf, nrows, ncols):
    def body(i, carry):
        for j in range(ncols // 16):
            buf[i, pl.ds(j * 16, 16)] = jnp.zeros((16,), jnp.float32)
        return carry
    lax.fori_loop(0, nrows, body, 0)


_CPT = _NCHUNK // _NW
_XTRA = _NCHUNK - _CPT * _NW


@functools.lru_cache(maxsize=None)
def _make_sc_scatter(F):
    stages = 2 if F == 128 else 1
    SR = _CPT // stages
    mesh = plsc.VectorSubcoreMesh(core_axis_name="c", subcore_axis_name="s")

    @functools.partial(
        pl.kernel,
        out_type=jax.ShapeDtypeStruct((_NC, _N, F), jnp.float32),
        mesh=mesh,
        compiler_params=pltpu.CompilerParams(use_tc_tiling_on_sc=False),
        scratch_types=[
            pltpu.VMEM((SR + 1, _CH), jnp.int32),
            pltpu.VMEM((SR + 1, _CH), jnp.int32),
            pltpu.VMEM((2, _CH, F), jnp.float32),
            pltpu.VMEM_SHARED((_N, F), jnp.float32),
            pltpu.SemaphoreType.DMA,
            pltpu.SemaphoreType.DMA,
        ],
    )
    def k(u_hbm, src_hbm, dst_hbm, out_hbm, src_v, dst_v, rows_v, acc_sh,
          sem0, sem1):
        cid = lax.axis_index("c")
        sid = lax.axis_index("s")
        wid = sid * _NC + cid

        _zero_rows(rows_v.at[0], _CH, F)
        zdescs = []
        for off, sz in _ZCHUNKS:
            zdescs.append(pltpu.async_copy(
                rows_v.at[0, pl.ds(0, sz)],
                acc_sh.at[pl.ds(sid * _RPT8 + off, sz)], sem0))
        @pl.when(sid == _NS - 1)
        def _():
            pltpu.async_copy(rows_v.at[0, pl.ds(0, _TAIL)],
                             acc_sh.at[pl.ds(_NS * _RPT8, _TAIL)], sem1)

        def load_stage(st):
            pltpu.sync_copy(src_hbm.at[pl.ds(wid * _CPT + st * SR, SR)],
                            src_v.at[pl.ds(0, SR)])
            pltpu.sync_copy(dst_hbm.at[pl.ds(wid * _CPT + st * SR, SR)],
                            dst_v.at[pl.ds(0, SR)])

        load_stage(0)
        @pl.when(wid < _XTRA)
        def _():
            pltpu.sync_copy(src_hbm.at[pl.ds(_NW * _CPT + wid, 1)],
                            src_v.at[pl.ds(SR, 1)])
            pltpu.sync_copy(dst_hbm.at[pl.ds(_NW * _CPT + wid, 1)],
                            dst_v.at[pl.ds(SR, 1)])
        for d in zdescs:
            d.wait()
        @pl.when(sid == _NS - 1)
        def _():
            pltpu.make_async_copy(rows_v.at[0, pl.ds(0, _TAIL)],
                                  acc_sh.at[pl.ds(_NS * _RPT8, _TAIL)],
                                  sem1).wait()
        plsc.subcore_barrier()

        sems = (sem0, sem1)

        def start(j, b):
            pltpu.async_copy(u_hbm.at[src_v.at[j]], rows_v.at[b], sems[b])

        def finish(j, b):
            pltpu.make_async_copy(u_hbm.at[src_v.at[j]], rows_v.at[b],
                                  sems[b]).wait()
            pltpu.sync_copy(rows_v.at[b], acc_sh.at[dst_v.at[j]], add=True)

        def pipeline():
            start(0, 0)

            def body(i, carry):
                j0 = 2 * i
                @pl.when(j0 + 1 < SR)
                def _():
                    start(j0 + 1, 1)
                finish(j0, 0)
                @pl.when(j0 + 2 < SR)
                def _():
                    start(j0 + 2, 0)
                @pl.when(j0 + 1 < SR)
                def _():
                    finish(j0 + 1, 1)
                return carry

            lax.fori_loop(0, (SR + 1) // 2, body, 0)

        for st in range(stages):
            if st > 0:
                load_stage(st)
            pipeline()

        @pl.when(wid < _XTRA)
        def _():
            start(SR, 0)
            finish(SR, 0)
        plsc.subcore_barrier()

        pltpu.sync_copy(acc_sh.at[pl.ds(sid * _RPT8, _RPT8)],
                        out_hbm.at[cid, pl.ds(sid * _RPT8, _RPT8)])
        @pl.when(sid == _NS - 1)
        def _():
            pltpu.sync_copy(acc_sh.at[pl.ds(_NS * _RPT8, _TAIL)],
                            out_hbm.at[cid, pl.ds(_NS * _RPT8, _TAIL)])

    return k


@functools.lru_cache(maxsize=None)
def _make_sc_degree():
    F = 16
    mesh = plsc.VectorSubcoreMesh(core_axis_name="c", subcore_axis_name="s")

    @functools.partial(
        pl.kernel,
        out_type=jax.ShapeDtypeStruct((_NC, _N, F), jnp.float32),
        mesh=mesh,
        compiler_params=pltpu.CompilerParams(use_tc_tiling_on_sc=False),
        scratch_types=[
            pltpu.VMEM((_CPT + 1, _CH), jnp.int32),
            pltpu.VMEM((_CH, F), jnp.float32),
            pltpu.VMEM_SHARED((_N, F), jnp.float32),
            pltpu.SemaphoreType.DMA,
        ],
    )
    def k(src_hbm, out_hbm, src_v, ones_v, acc_sh, sem):
        cid = lax.axis_index("c")
        sid = lax.axis_index("s")
        wid = sid * _NC + cid

        _zero_rows(ones_v, _CH, F)
        for off, sz in _ZCHUNKS:
            pltpu.sync_copy(ones_v.at[pl.ds(0, sz)],
                            acc_sh.at[pl.ds(sid * _RPT8 + off, sz)])
        @pl.when(sid == _NS - 1)
        def _():
            pltpu.sync_copy(ones_v.at[pl.ds(0, _TAIL)],
                            acc_sh.at[pl.ds(_NS * _RPT8, _TAIL)])

        pltpu.sync_copy(src_hbm.at[pl.ds(wid * _CPT, _CPT)],
                        src_v.at[pl.ds(0, _CPT)])
        @pl.when(wid < _XTRA)
        def _():
            pltpu.sync_copy(src_hbm.at[pl.ds(_NW * _CPT + wid, 1)],
                            src_v.at[pl.ds(_CPT, 1)])
        plsc.subcore_barrier()

        def fill(i, carry):
            ones_v[i, pl.ds(0, 16)] = jnp.ones((16,), jnp.float32)
            return carry
        lax.fori_loop(0, _CH, fill, 0)

        def body(i, carry):
            pltpu.sync_copy(ones_v, acc_sh.at[src_v.at[i]], add=True)
            return carry

        lax.fori_loop(0, _CPT, body, 0)
        @pl.when(wid < _XTRA)
        def _():
            pltpu.sync_copy(ones_v, acc_sh.at[src_v.at[_CPT]], add=True)
        plsc.subcore_barrier()

        pltpu.sync_copy(acc_sh.at[pl.ds(sid * _RPT8, _RPT8)],
                        out_hbm.at[cid, pl.ds(sid * _RPT8, _RPT8)])
        @pl.when(sid == _NS - 1)
        def _():
            pltpu.sync_copy(acc_sh.at[pl.ds(_NS * _RPT8, _TAIL)],
                            out_hbm.at[cid, pl.ds(_NS * _RPT8, _TAIL)])

    return k


def _rb(shape, idx=None):
    if idx is None:
        idx = lambda i: (i,) + (0,) * (len(shape) - 1)
    return pl.BlockSpec(shape, idx)


@functools.lru_cache(maxsize=None)
def _make_prep_clen(Fin, Fo, S):
    def body(degp_ref, x_ref, w_ref, dis_ref, z_ref, u_ref):
        d = degp_ref[0] + degp_ref[1]
        deg = d[:, 0:1]
        dis = jnp.where(deg > 0.0, lax.rsqrt(jnp.maximum(deg, 1e-30)), 0.0)
        dis_ref[...] = dis
        x = x_ref[...]
        for k in range(S):
            zk = jnp.dot(x, w_ref[k], preferred_element_type=jnp.float32)
            z_ref[k] = zk
        u_ref[...] = dis * zk

    return pl.pallas_call(
        body,
        grid=(_NB,),
        in_specs=[
            pl.BlockSpec((_NC, _BR, 16), lambda i: (0, i, 0)),
            _rb((_BR, Fin)),
            pl.BlockSpec((S, Fin, Fo), lambda i: (0, 0, 0)),
        ],
        out_specs=[_rb((_BR, 1)),
                   pl.BlockSpec((S, _BR, Fo), lambda i: (0, i, 0)),
                   _rb((_BR, Fo))],
        out_shape=[
            jax.ShapeDtypeStruct((_N, 1), jnp.float32),
            jax.ShapeDtypeStruct((S, _N, Fo), jnp.float32),
            jax.ShapeDtypeStruct((_N, Fo), jnp.float32),
        ],
    )


@functools.lru_cache(maxsize=None)
def _make_clen_step(zk, bprev_from_z, bpp_kind, Fo):
    def body(*refs):
        it = list(refs)
        sp_ref = it.pop(0)
        bp_ref = it.pop(0)
        bpp_ref = it.pop(0) if bpp_kind != 'none' else None
        z_ref, dis_ref, lam_ref, b_ref, u_ref = it
        s = 2.0 / lam_ref[0]
        dis = dis_ref[...]
        bp = bp_ref[0] if bprev_from_z else bp_ref[...]
        a = (-s) * (dis * (sp_ref[0] + sp_ref[1])) + (s - 1.0) * bp
        b = 2.0 * a + z_ref[0]
        if bpp_ref is not None:
            b = b - (bpp_ref[0] if bpp_kind == 'z' else bpp_ref[...])
        b_ref[...] = b
        u_ref[...] = dis * b

    def zblk(k):
        return pl.BlockSpec((1, _BR, Fo), lambda i, k=k: (k, i, 0))

    in_specs = [pl.BlockSpec((_NC, _BR, Fo), lambda i: (0, i, 0))]
    in_specs.append(zblk(4) if bprev_from_z else _rb((_BR, Fo)))
    if bpp_kind == 'z':
        in_specs.append(zblk(4))
    elif bpp_kind == 'arr':
        in_specs.append(_rb((_BR, Fo)))
    in_specs += [zblk(zk), _rb((_BR, 1)),
                 pl.BlockSpec(memory_space=pltpu.SMEM)]

    return pl.pallas_call(
        body, grid=(_NB,), in_specs=in_specs,
        out_specs=[_rb((_BR, Fo)), _rb((_BR, Fo))],
        out_shape=[jax.ShapeDtypeStruct((_N, Fo), jnp.float32),
                   jax.ShapeDtypeStruct((_N, Fo), jnp.float32)],
    )


@functools.lru_cache(maxsize=None)
def _make_clen_finish_start(Fo, Fo2):
    def body(sp_ref, b1_ref, b2_ref, z_ref, dis_ref, lam_ref, bias_ref,
             w0_ref, h_ref, u_ref, acc_ref):
        s = 2.0 / lam_ref[0]
        dis = dis_ref[...]
        a = (-s) * (dis * (sp_ref[0] + sp_ref[1])) \
            + (s - 1.0) * b1_ref[...]
        out = a - b2_ref[...] + z_ref[0]
        h = jax.nn.relu(out + bias_ref[...])
        h_ref[...] = h
        u_ref[...] = dis * h
        acc_ref[...] = jnp.dot(h, w0_ref[...],
                               preferred_element_type=jnp.float32)

    return pl.pallas_call(
        body,
        grid=(_NB,),
        in_specs=[
            pl.BlockSpec((_NC, _BR, Fo), lambda i: (0, i, 0)),
            _rb((_BR, Fo)),
            _rb((_BR, Fo)),
            pl.BlockSpec((1, _BR, Fo), lambda i: (0, i, 0)),
            _rb((_BR, 1)),
            pl.BlockSpec(memory_space=pltpu.SMEM),
            pl.BlockSpec((1, Fo), lambda i: (0, 0)),
            pl.BlockSpec((Fo, Fo2), lambda i: (0, 0)),
        ],
        out_specs=[_rb((_BR, Fo)), _rb((_BR, Fo)), _rb((_BR, Fo2))],
        out_shape=[
            jax.ShapeDtypeStruct((_N, Fo), jnp.float32),
            jax.ShapeDtypeStruct((_N, Fo), jnp.float32),
            jax.ShapeDtypeStruct((_N, Fo2), jnp.float32),
        ],
    )


@functools.lru_cache(maxsize=None)
def _make_step(m, Fin, Fout, emit_u):
    def body(*refs):
        if m == 2:
            (sp_ref, ta_ref, tb_ref, dis_ref, w_ref, lam_ref, acc_ref,
             *outs) = refs
        else:
            sp_ref, ta_ref, dis_ref, w_ref, lam_ref, acc_ref, *outs = refs
            tb_ref = None
        new_ref, acco_ref = outs[0], outs[1]
        s = 2.0 / lam_ref[0]
        dis = dis_ref[...]
        sm = sp_ref[0] + sp_ref[1]
        new = (-m * s) * (dis * sm) + (m * (s - 1.0)) * ta_ref[...]
        if m == 2:
            new = new - tb_ref[...]
        new_ref[...] = new
        acco_ref[...] = acc_ref[...] + jnp.dot(
            new, w_ref[...], preferred_element_type=jnp.float32)
        if emit_u:
            outs[2][...] = dis * new

    in_specs = [pl.BlockSpec((_NC, _BR, Fin), lambda i: (0, i, 0)),
                _rb((_BR, Fin))]
    if m == 2:
        in_specs.append(_rb((_BR, Fin)))
    in_specs += [
        _rb((_BR, 1)),
        pl.BlockSpec((Fin, Fout), lambda i: (0, 0)),
        pl.BlockSpec(memory_space=pltpu.SMEM),
        _rb((_BR, Fout)),
    ]
    out_specs = [_rb((_BR, Fin)), _rb((_BR, Fout))]
    out_shape = [jax.ShapeDtypeStruct((_N, Fin), jnp.float32),
                 jax.ShapeDtypeStruct((_N, Fout), jnp.float32)]
    if emit_u:
        out_specs.append(_rb((_BR, Fin)))
        out_shape.append(jax.ShapeDtypeStruct((_N, Fin), jnp.float32))

    return pl.pallas_call(
        body, grid=(_NB,), in_specs=in_specs,
        out_specs=out_specs, out_shape=out_shape,
    )


@functools.lru_cache(maxsize=None)
def _make_step_start(Fin, Fout, Fo2):
    def body(sp_ref, ta_ref, tb_ref, dis_ref, w4_ref, w0_ref, lam_ref,
             acc_ref, b_ref, h_ref, u_ref, acco_ref):
        s = 2.0 / lam_ref[0]
        dis = dis_ref[...]
        sm = sp_ref[0] + sp_ref[1]
        new = (-2.0 * s) * (dis * sm) + (2.0 * (s - 1.0)) * ta_ref[...] \
            - tb_ref[...]
        acc4 = acc_ref[...] + jnp.dot(new, w4_ref[...],
                                      preferred_element_type=jnp.float32)
        h = jax.nn.relu(acc4 + b_ref[...])
        h_ref[...] = h
        u_ref[...] = dis * h
        acco_ref[...] = jnp.dot(h, w0_ref[...],
                                preferred_element_type=jnp.float32)

    return pl.pallas_call(
        body,
        grid=(_NB,),
        in_specs=[
            pl.BlockSpec((_NC, _BR, Fin), lambda i: (0, i, 0)),
            _rb((_BR, Fin)),
            _rb((_BR, Fin)),
            _rb((_BR, 1)),
            pl.BlockSpec((Fin, Fout), lambda i: (0, 0)),
            pl.BlockSpec((Fout, Fo2), lambda i: (0, 0)),
            pl.BlockSpec(memory_space=pltpu.SMEM),
            _rb((_BR, Fout)),
            pl.BlockSpec((1, Fout), lambda i: (0, 0)),
        ],
        out_specs=[_rb((_BR, Fout)), _rb((_BR, Fout)), _rb((_BR, Fo2))],
        out_shape=[
            jax.ShapeDtypeStruct((_N, Fout), jnp.float32),
            jax.ShapeDtypeStruct((_N, Fout), jnp.float32),
            jax.ShapeDtypeStruct((_N, Fo2), jnp.float32),
        ],
    )


@functools.lru_cache(maxsize=None)
def _make_step_pool(Fin, Fout):
    def body(sp_ref, ta_ref, tb_ref, dis_ref, w4_ref, lam_ref, acc_ref,
             b_ref, w1_ref, b1_ref, w2_ref, b2_ref, out_ref, sum_ref):
        i = pl.program_id(0)
        s = 2.0 / lam_ref[0]
        dis = dis_ref[...]
        sm = sp_ref[0] + sp_ref[1]
        new = (-2.0 * s) * (dis * sm) + (2.0 * (s - 1.0)) * ta_ref[...] \
            - tb_ref[...]
        acc4 = acc_ref[...] + jnp.dot(new, w4_ref[...],
                                      preferred_element_type=jnp.float32)
        h = jax.nn.relu(acc4 + b_ref[...])
        part = jnp.sum(h, axis=0, keepdims=True)

        @pl.when(i == 0)
        def _():
            sum_ref[...] = part

        @pl.when(i > 0)
        def _():
            sum_ref[...] = sum_ref[...] + part

        @pl.when(i == _NB - 1)
        def _():
            pooled = sum_ref[...] / jnp.float32(_N)
            t = jax.nn.relu(
                jnp.dot(pooled, w1_ref[...],
                        preferred_element_type=jnp.float32) + b1_ref[...])
            out_ref[...] = jnp.dot(
                t, w2_ref[...], preferred_element_type=jnp.float32) \
                + b2_ref[...]

    return pl.pallas_call(
        body,
        grid=(_NB,),
        in_specs=[
            pl.BlockSpec((_NC, _BR, Fin), lambda i: (0, i, 0)),
            _rb((_BR, Fin)),
            _rb((_BR, Fin)),
            _rb((_BR, 1)),
            pl.BlockSpec((Fin, Fout), lambda i: (0, 0)),
            pl.BlockSpec(memory_space=pltpu.SMEM),
            _rb((_BR, Fout)),
            pl.BlockSpec((1, Fout), lambda i: (0, 0)),
            pl.BlockSpec((Fout, 10), lambda i: (0, 0)),
            pl.BlockSpec((1, 10), lambda i: (0, 0)),
            pl.BlockSpec((10, 10), lambda i: (0, 0)),
            pl.BlockSpec((1, 10), lambda i: (0, 0)),
        ],
        out_specs=pl.BlockSpec((1, 10), lambda i: (0, 0)),
        out_shape=jax.ShapeDtypeStruct((1, 10), jnp.float32),
        scratch_shapes=[pltpu.VMEM((1, Fout), jnp.float32)],
    )


def kernel(x, edge_index, lambda_max, batch, W1, b1, W2, b2, W3, b3,
           fc1_w, fc1_b, fc2_w, fc2_b):
    src = edge_index[0].reshape(_NCHUNK, _CH)
    dst = edge_index[1].reshape(_NCHUNK, _CH)
    lam = lambda_max.reshape((1,)).astype(jnp.float32)

    degp = _make_sc_degree()(src)

    dis, z, u = _make_prep_clen(128, 32, 5)(degp, x, W1)
    scat32 = _make_sc_scatter(32)
    sp = scat32(u, src, dst)
    cb3, u = _make_clen_step(3, True, 'none', 32)(sp, z, z, dis, lam)
    sp = scat32(u, src, dst)
    cb2, u = _make_clen_step(2, False, 'z', 32)(sp, cb3, z, z, dis, lam)
    sp = scat32(u, src, dst)
    cb1, u = _make_clen_step(1, False, 'arr', 32)(sp, cb2, cb3, z, dis, lam)
    sp = scat32(u, src, dst)
    h, u, acc = _make_clen_finish_start(32, 64)(
        sp, cb1, cb2, z, dis, lam, b1.reshape(1, -1), W2[0])

    def run_3hops(u, acc, t0, dis, W, Fin, Fout):
        scat = _make_sc_scatter(Fin)
        sp = scat(u, src, dst)
        t1, acc, u = _make_step(1, Fin, Fout, True)(sp, t0, dis, W[1], lam,
                                                    acc)
        for k in (2, 3):
            sp = scat(u, src, dst)
            t2, acc, u = _make_step(2, Fin, Fout, True)(sp, t1, t0, dis,
                                                        W[k], lam, acc)
            t0, t1 = t1, t2
        sp = scat(u, src, dst)
        return sp, t1, t0, acc

    sp, t1, t0, acc = run_3hops(u, acc, h, dis, W2, 32, 64)
    h, u, acc = _make_step_start(32, 64, 64)(
        sp, t1, t0, dis, W2[4], W3[0], lam, acc, b2.reshape(1, -1))

    sp, t1, t0, acc = run_3hops(u, acc, h, dis, W3, 64, 64)
    return _make_step_pool(64, 64)(
        sp, t1, t0, dis, W3[4], lam, acc, b3.reshape(1, -1),
        fc1_w, fc1_b.reshape(1, -1), fc2_w, fc2_b.reshape(1, -1))


# speedup vs baseline: 1.2996x; 1.0629x over previous
import functools

import jax
import jax.numpy as jnp
from jax import lax
from jax.experimental import pallas as pl
from jax.experimental.pallas import tpu as pltpu
from jax.experimental.pallas import tpu_sc as plsc

_N = 10000
_E = 320000
_CH = 128
_NCHUNK = _E // _CH
_NC = 2
_NS = 16
_NW = _NC * _NS
_RPT = _N // _NS

_BR = 1000
_NB = _N // _BR

_RPT8 = 624
_TAIL = _N - _NS * _RPT8
_ZCHUNKS = ((0, 128), (128, 128), (256, 128), (384, 128), (512, 112))


def _zero_rows(buf, nrows, ncols):
    def body(i, carry):
        for j in range(ncols // 16):
            buf[i, pl.ds(j * 16, 16)] = jnp.zeros((16,), jnp.float32)
        return carry
    lax.fori_loop(0, nrows, body, 0)


_CPT = _NCHUNK // _NW
_XTRA = _NCHUNK - _CPT * _NW


@functools.lru_cache(maxsize=None)
def _make_sc_scatter(F):
    SR = _CPT
    mesh = plsc.VectorSubcoreMesh(core_axis_name="c", subcore_axis_name="s")

    @functools.partial(
        pl.kernel,
        out_type=jax.ShapeDtypeStruct((_NC, _N, F), jnp.float32),
        mesh=mesh,
        compiler_params=pltpu.CompilerParams(use_tc_tiling_on_sc=False),
        scratch_types=[
            pltpu.VMEM((SR + 1, _CH), jnp.int32),
            pltpu.VMEM((SR + 1, _CH), jnp.int32),
            pltpu.VMEM((4, _CH, F), jnp.float32),
            pltpu.VMEM_SHARED((_N, F), jnp.float32),
            pltpu.SemaphoreType.DMA,
            pltpu.SemaphoreType.DMA,
            pltpu.SemaphoreType.DMA,
            pltpu.SemaphoreType.DMA,
            pltpu.SemaphoreType.DMA,
            pltpu.SemaphoreType.DMA,
            pltpu.SemaphoreType.DMA,
            pltpu.SemaphoreType.DMA,
        ],
    )
    def k(u_hbm, src_hbm, dst_hbm, out_hbm, src_v, dst_v, rows_v, acc_sh,
          g0, g1, g2, g3, s0, s1, s2, s3):
        cid = lax.axis_index("c")
        sid = lax.axis_index("s")
        wid = sid * _NC + cid
        gsem = (g0, g1, g2, g3)
        ssem = (s0, s1, s2, s3)

        _zero_rows(rows_v.at[0], _CH, F)
        zdescs = []
        for off, sz in _ZCHUNKS:
            zdescs.append(pltpu.async_copy(
                rows_v.at[0, pl.ds(0, sz)],
                acc_sh.at[pl.ds(sid * _RPT8 + off, sz)], g0))
        @pl.when(sid == _NS - 1)
        def _():
            pltpu.async_copy(rows_v.at[0, pl.ds(0, _TAIL)],
                             acc_sh.at[pl.ds(_NS * _RPT8, _TAIL)], g1)

        pltpu.sync_copy(src_hbm.at[pl.ds(wid * _CPT, SR)],
                        src_v.at[pl.ds(0, SR)])
        pltpu.sync_copy(dst_hbm.at[pl.ds(wid * _CPT, SR)],
                        dst_v.at[pl.ds(0, SR)])
        @pl.when(wid < _XTRA)
        def _():
            pltpu.sync_copy(src_hbm.at[pl.ds(_NW * _CPT + wid, 1)],
                            src_v.at[pl.ds(SR, 1)])
            pltpu.sync_copy(dst_hbm.at[pl.ds(_NW * _CPT + wid, 1)],
                            dst_v.at[pl.ds(SR, 1)])
        for d in zdescs:
            d.wait()
        @pl.when(sid == _NS - 1)
        def _():
            pltpu.make_async_copy(rows_v.at[0, pl.ds(0, _TAIL)],
                                  acc_sh.at[pl.ds(_NS * _RPT8, _TAIL)],
                                  g1).wait()
        plsc.subcore_barrier()

        def sg(j, b):
            pltpu.async_copy(u_hbm.at[src_v.at[j]], rows_v.at[b], gsem[b])

        def wg(j, b):
            pltpu.make_async_copy(u_hbm.at[src_v.at[j]], rows_v.at[b],
                                  gsem[b]).wait()

        def ss(j, b):
            pltpu.async_copy(rows_v.at[b], acc_sh.at[dst_v.at[j]], ssem[b],
                             add=True)

        def ws(j, b):
            pltpu.make_async_copy(rows_v.at[b], acc_sh.at[dst_v.at[j]],
                                  ssem[b]).wait()

        sg(0, 0)
        sg(1, 1)

        def body(i, carry):
            for b in range(4):
                j = 4 * i + b
                wg(j, b)
                ss(j, b)
                bn = (b + 2) % 4
                if b >= 2:
                    ws(j - 2, bn)
                    sg(j + 2, bn)
                else:
                    @pl.when(i > 0)
                    def _(j=j, bn=bn):
                        ws(j - 2, bn)
                    sg(j + 2, bn)
            return carry

        lax.fori_loop(0, SR // 4, body, 0)
        wg(SR - 2, 0)
        ss(SR - 2, 0)
        ws(SR - 4, 2)
        wg(SR - 1, 1)
        ss(SR - 1, 1)
        ws(SR - 3, 3)
        ws(SR - 2, 0)
        ws(SR - 1, 1)

        @pl.when(wid < _XTRA)
        def _():
            sg(SR, 2)
            wg(SR, 2)
            ss(SR, 2)
            ws(SR, 2)
        plsc.subcore_barrier()

        pltpu.sync_copy(acc_sh.at[pl.ds(sid * _RPT8, _RPT8)],
                        out_hbm.at[cid, pl.ds(sid * _RPT8, _RPT8)])
        @pl.when(sid == _NS - 1)
        def _():
            pltpu.sync_copy(acc_sh.at[pl.ds(_NS * _RPT8, _TAIL)],
                            out_hbm.at[cid, pl.ds(_NS * _RPT8, _TAIL)])

    return k


@functools.lru_cache(maxsize=None)
def _make_sc_degree():
    F = 16
    mesh = plsc.VectorSubcoreMesh(core_axis_name="c", subcore_axis_name="s")

    @functools.partial(
        pl.kernel,
        out_type=jax.ShapeDtypeStruct((_NC, _N, F), jnp.float32),
        mesh=mesh,
        compiler_params=pltpu.CompilerParams(use_tc_tiling_on_sc=False),
        scratch_types=[
            pltpu.VMEM((_CPT + 1, _CH), jnp.int32),
            pltpu.VMEM((_CH, F), jnp.float32),
            pltpu.VMEM_SHARED((_N, F), jnp.float32),
            pltpu.SemaphoreType.DMA,
        ],
    )
    def k(src_hbm, out_hbm, src_v, ones_v, acc_sh, sem):
        cid = lax.axis_index("c")
        sid = lax.axis_index("s")
        wid = sid * _NC + cid

        _zero_rows(ones_v, _CH, F)
        for off, sz in _ZCHUNKS:
            pltpu.sync_copy(ones_v.at[pl.ds(0, sz)],
                            acc_sh.at[pl.ds(sid * _RPT8 + off, sz)])
        @pl.when(sid == _NS - 1)
        def _():
            pltpu.sync_copy(ones_v.at[pl.ds(0, _TAIL)],
                            acc_sh.at[pl.ds(_NS * _RPT8, _TAIL)])

        pltpu.sync_copy(src_hbm.at[pl.ds(wid * _CPT, _CPT)],
                        src_v.at[pl.ds(0, _CPT)])
        @pl.when(wid < _XTRA)
        def _():
            pltpu.sync_copy(src_hbm.at[pl.ds(_NW * _CPT + wid, 1)],
                            src_v.at[pl.ds(_CPT, 1)])
        plsc.subcore_barrier()

        def fill(i, carry):
            ones_v[i, pl.ds(0, 16)] = jnp.ones((16,), jnp.float32)
            return carry
        lax.fori_loop(0, _CH, fill, 0)

        def body(i, carry):
            pltpu.sync_copy(ones_v, acc_sh.at[src_v.at[i]], add=True)
            return carry

        lax.fori_loop(0, _CPT, body, 0)
        @pl.when(wid < _XTRA)
        def _():
            pltpu.sync_copy(ones_v, acc_sh.at[src_v.at[_CPT]], add=True)
        plsc.subcore_barrier()

        pltpu.sync_copy(acc_sh.at[pl.ds(sid * _RPT8, _RPT8)],
                        out_hbm.at[cid, pl.ds(sid * _RPT8, _RPT8)])
        @pl.when(sid == _NS - 1)
        def _():
            pltpu.sync_copy(acc_sh.at[pl.ds(_NS * _RPT8, _TAIL)],
                            out_hbm.at[cid, pl.ds(_NS * _RPT8, _TAIL)])

    return k


def _rb(shape, idx=None):
    if idx is None:
        idx = lambda i: (i,) + (0,) * (len(shape) - 1)
    return pl.BlockSpec(shape, idx)


@functools.lru_cache(maxsize=None)
def _make_prep_clen(Fin, Fo, S):
    def body(degp_ref, x_ref, w_ref, dis_ref, z_ref, u_ref):
        d = degp_ref[0] + degp_ref[1]
        deg = d[:, 0:1]
        dis = jnp.where(deg > 0.0, lax.rsqrt(jnp.maximum(deg, 1e-30)), 0.0)
        dis_ref[...] = dis
        x = x_ref[...]
        for k in range(S):
            zk = jnp.dot(x, w_ref[k], preferred_element_type=jnp.float32)
            z_ref[k] = zk
        u_ref[...] = dis * zk

    return pl.pallas_call(
        body,
        grid=(_NB,),
        in_specs=[
            pl.BlockSpec((_NC, _BR, 16), lambda i: (0, i, 0)),
            _rb((_BR, Fin)),
            pl.BlockSpec((S, Fin, Fo), lambda i: (0, 0, 0)),
        ],
        out_specs=[_rb((_BR, 1)),
                   pl.BlockSpec((S, _BR, Fo), lambda i: (0, i, 0)),
                   _rb((_BR, Fo))],
        out_shape=[
            jax.ShapeDtypeStruct((_N, 1), jnp.float32),
            jax.ShapeDtypeStruct((S, _N, Fo), jnp.float32),
            jax.ShapeDtypeStruct((_N, Fo), jnp.float32),
        ],
    )


@functools.lru_cache(maxsize=None)
def _make_clen_step(zk, bprev_from_z, bpp_kind, Fo):
    def body(*refs):
        it = list(refs)
        sp_ref = it.pop(0)
        bp_ref = it.pop(0)
        bpp_ref = it.pop(0) if bpp_kind != 'none' else None
        z_ref, dis_ref, lam_ref, b_ref, u_ref = it
        s = 2.0 / lam_ref[0]
        dis = dis_ref[...]
        bp = bp_ref[0] if bprev_from_z else bp_ref[...]
        a = (-s) * (dis * (sp_ref[0] + sp_ref[1])) + (s - 1.0) * bp
        b = 2.0 * a + z_ref[0]
        if bpp_ref is not None:
            b = b - (bpp_ref[0] if bpp_kind == 'z' else bpp_ref[...])
        b_ref[...] = b
        u_ref[...] = dis * b

    def zblk(k):
        return pl.BlockSpec((1, _BR, Fo), lambda i, k=k: (k, i, 0))

    in_specs = [pl.BlockSpec((_NC, _BR, Fo), lambda i: (0, i, 0))]
    in_specs.append(zblk(4) if bprev_from_z else _rb((_BR, Fo)))
    if bpp_kind == 'z':
        in_specs.append(zblk(4))
    elif bpp_kind == 'arr':
        in_specs.append(_rb((_BR, Fo)))
    in_specs += [zblk(zk), _rb((_BR, 1)),
                 pl.BlockSpec(memory_space=pltpu.SMEM)]

    return pl.pallas_call(
        body, grid=(_NB,), in_specs=in_specs,
        out_specs=[_rb((_BR, Fo)), _rb((_BR, Fo))],
        out_shape=[jax.ShapeDtypeStruct((_N, Fo), jnp.float32),
                   jax.ShapeDtypeStruct((_N, Fo), jnp.float32)],
    )


@functools.lru_cache(maxsize=None)
def _make_clen_finish_start(Fo, Fo2):
    def body(sp_ref, b1_ref, b2_ref, z_ref, dis_ref, lam_ref, bias_ref,
             w0_ref, h_ref, u_ref, acc_ref):
        s = 2.0 / lam_ref[0]
        dis = dis_ref[...]
        a = (-s) * (dis * (sp_ref[0] + sp_ref[1])) \
            + (s - 1.0) * b1_ref[...]
        out = a - b2_ref[...] + z_ref[0]
        h = jax.nn.relu(out + bias_ref[...])
        h_ref[...] = h
        u_ref[...] = dis * h
        acc_ref[...] = jnp.dot(h, w0_ref[...],
                               preferred_element_type=jnp.float32)

    return pl.pallas_call(
        body,
        grid=(_NB,),
        in_specs=[
            pl.BlockSpec((_NC, _BR, Fo), lambda i: (0, i, 0)),
            _rb((_BR, Fo)),
            _rb((_BR, Fo)),
            pl.BlockSpec((1, _BR, Fo), lambda i: (0, i, 0)),
            _rb((_BR, 1)),
            pl.BlockSpec(memory_space=pltpu.SMEM),
            pl.BlockSpec((1, Fo), lambda i: (0, 0)),
            pl.BlockSpec((Fo, Fo2), lambda i: (0, 0)),
        ],
        out_specs=[_rb((_BR, Fo)), _rb((_BR, Fo)), _rb((_BR, Fo2))],
        out_shape=[
            jax.ShapeDtypeStruct((_N, Fo), jnp.float32),
            jax.ShapeDtypeStruct((_N, Fo), jnp.float32),
            jax.ShapeDtypeStruct((_N, Fo2), jnp.float32),
        ],
    )


@functools.lru_cache(maxsize=None)
def _make_step(m, Fin, Fout, emit_u):
    def body(*refs):
        if m == 2:
            (sp_ref, ta_ref, tb_ref, dis_ref, w_ref, lam_ref, acc_ref,
             *outs) = refs
        else:
            sp_ref, ta_ref, dis_ref, w_ref, lam_ref, acc_ref, *outs = refs
            tb_ref = None
        new_ref, acco_ref = outs[0], outs[1]
        s = 2.0 / lam_ref[0]
        dis = dis_ref[...]
        sm = sp_ref[0] + sp_ref[1]
        new = (-m * s) * (dis * sm) + (m * (s - 1.0)) * ta_ref[...]
        if m == 2:
            new = new - tb_ref[...]
        new_ref[...] = new
        acco_ref[...] = acc_ref[...] + jnp.dot(
            new, w_ref[...], preferred_element_type=jnp.float32)
        if emit_u:
            outs[2][...] = dis * new

    in_specs = [pl.BlockSpec((_NC, _BR, Fin), lambda i: (0, i, 0)),
                _rb((_BR, Fin))]
    if m == 2:
        in_specs.append(_rb((_BR, Fin)))
    in_specs += [
        _rb((_BR, 1)),
        pl.BlockSpec((Fin, Fout), lambda i: (0, 0)),
        pl.BlockSpec(memory_space=pltpu.SMEM),
        _rb((_BR, Fout)),
    ]
    out_specs = [_rb((_BR, Fin)), _rb((_BR, Fout))]
    out_shape = [jax.ShapeDtypeStruct((_N, Fin), jnp.float32),
                 jax.ShapeDtypeStruct((_N, Fout), jnp.float32)]
    if emit_u:
        out_specs.append(_rb((_BR, Fin)))
        out_shape.append(jax.ShapeDtypeStruct((_N, Fin), jnp.float32))

    return pl.pallas_call(
        body, grid=(_NB,), in_specs=in_specs,
        out_specs=out_specs, out_shape=out_shape,
    )


@functools.lru_cache(maxsize=None)
def _make_step_start(Fin, Fout, Fo2):
    def body(sp_ref, ta_ref, tb_ref, dis_ref, w4_ref, w0_ref, lam_ref,
             acc_ref, b_ref, h_ref, u_ref, acco_ref):
        s = 2.0 / lam_ref[0]
        dis = dis_ref[...]
        sm = sp_ref[0] + sp_ref[1]
        new = (-2.0 * s) * (dis * sm) + (2.0 * (s - 1.0)) * ta_ref[...] \
            - tb_ref[...]
        acc4 = acc_ref[...] + jnp.dot(new, w4_ref[...],
                                      preferred_element_type=jnp.float32)
        h = jax.nn.relu(acc4 + b_ref[...])
        h_ref[...] = h
        u_ref[...] = dis * h
        acco_ref[...] = jnp.dot(h, w0_ref[...],
                                preferred_element_type=jnp.float32)

    return pl.pallas_call(
        body,
        grid=(_NB,),
        in_specs=[
            pl.BlockSpec((_NC, _BR, Fin), lambda i: (0, i, 0)),
            _rb((_BR, Fin)),
            _rb((_BR, Fin)),
            _rb((_BR, 1)),
            pl.BlockSpec((Fin, Fout), lambda i: (0, 0)),
            pl.BlockSpec((Fout, Fo2), lambda i: (0, 0)),
            pl.BlockSpec(memory_space=pltpu.SMEM),
            _rb((_BR, Fout)),
            pl.BlockSpec((1, Fout), lambda i: (0, 0)),
        ],
        out_specs=[_rb((_BR, Fout)), _rb((_BR, Fout)), _rb((_BR, Fo2))],
        out_shape=[
            jax.ShapeDtypeStruct((_N, Fout), jnp.float32),
            jax.ShapeDtypeStruct((_N, Fout), jnp.float32),
            jax.ShapeDtypeStruct((_N, Fo2), jnp.float32),
        ],
    )


@functools.lru_cache(maxsize=None)
def _make_step_pool(Fin, Fout):
    def body(sp_ref, ta_ref, tb_ref, dis_ref, w4_ref, lam_ref, acc_ref,
             b_ref, w1_ref, b1_ref, w2_ref, b2_ref, out_ref, sum_ref):
        i = pl.program_id(0)
        s = 2.0 / lam_ref[0]
        dis = dis_ref[...]
        sm = sp_ref[0] + sp_ref[1]
        new = (-2.0 * s) * (dis * sm) + (2.0 * (s - 1.0)) * ta_ref[...] \
            - tb_ref[...]
        acc4 = acc_ref[...] + jnp.dot(new, w4_ref[...],
                                      preferred_element_type=jnp.float32)
        h = jax.nn.relu(acc4 + b_ref[...])
        part = jnp.sum(h, axis=0, keepdims=True)

        @pl.when(i == 0)
        def _():
            sum_ref[...] = part

        @pl.when(i > 0)
        def _():
            sum_ref[...] = sum_ref[...] + part

        @pl.when(i == _NB - 1)
        def _():
            pooled = sum_ref[...] / jnp.float32(_N)
            t = jax.nn.relu(
                jnp.dot(pooled, w1_ref[...],
                        preferred_element_type=jnp.float32) + b1_ref[...])
            out_ref[...] = jnp.dot(
                t, w2_ref[...], preferred_element_type=jnp.float32) \
                + b2_ref[...]

    return pl.pallas_call(
        body,
        grid=(_NB,),
        in_specs=[
            pl.BlockSpec((_NC, _BR, Fin), lambda i: (0, i, 0)),
            _rb((_BR, Fin)),
            _rb((_BR, Fin)),
            _rb((_BR, 1)),
            pl.BlockSpec((Fin, Fout), lambda i: (0, 0)),
            pl.BlockSpec(memory_space=pltpu.SMEM),
            _rb((_BR, Fout)),
            pl.BlockSpec((1, Fout), lambda i: (0, 0)),
            pl.BlockSpec((Fout, 10), lambda i: (0, 0)),
            pl.BlockSpec((1, 10), lambda i: (0, 0)),
            pl.BlockSpec((10, 10), lambda i: (0, 0)),
            pl.BlockSpec((1, 10), lambda i: (0, 0)),
        ],
        out_specs=pl.BlockSpec((1, 10), lambda i: (0, 0)),
        out_shape=jax.ShapeDtypeStruct((1, 10), jnp.float32),
        scratch_shapes=[pltpu.VMEM((1, Fout), jnp.float32)],
    )


def kernel(x, edge_index, lambda_max, batch, W1, b1, W2, b2, W3, b3,
           fc1_w, fc1_b, fc2_w, fc2_b):
    src = edge_index[0].reshape(_NCHUNK, _CH)
    dst = edge_index[1].reshape(_NCHUNK, _CH)
    lam = lambda_max.reshape((1,)).astype(jnp.float32)

    degp = _make_sc_degree()(src)

    dis, z, u = _make_prep_clen(128, 32, 5)(degp, x, W1)
    scat32 = _make_sc_scatter(32)
    sp = scat32(u, src, dst)
    cb3, u = _make_clen_step(3, True, 'none', 32)(sp, z, z, dis, lam)
    sp = scat32(u, src, dst)
    cb2, u = _make_clen_step(2, False, 'z', 32)(sp, cb3, z, z, dis, lam)
    sp = scat32(u, src, dst)
    cb1, u = _make_clen_step(1, False, 'arr', 32)(sp, cb2, cb3, z, dis, lam)
    sp = scat32(u, src, dst)
    h, u, acc = _make_clen_finish_start(32, 64)(
        sp, cb1, cb2, z, dis, lam, b1.reshape(1, -1), W2[0])

    def run_3hops(u, acc, t0, dis, W, Fin, Fout):
        scat = _make_sc_scatter(Fin)
        sp = scat(u, src, dst)
        t1, acc, u = _make_step(1, Fin, Fout, True)(sp, t0, dis, W[1], lam,
                                                    acc)
        for k in (2, 3):
            sp = scat(u, src, dst)
            t2, acc, u = _make_step(2, Fin, Fout, True)(sp, t1, t0, dis,
                                                        W[k], lam, acc)
            t0, t1 = t1, t2
        sp = scat(u, src, dst)
        return sp, t1, t0, acc

    sp, t1, t0, acc = run_3hops(u, acc, h, dis, W2, 32, 64)
    h, u, acc = _make_step_start(32, 64, 64)(
        sp, t1, t0, dis, W2[4], W3[0], lam, acc, b2.reshape(1, -1))

    sp, t1, t0, acc = run_3hops(u, acc, h, dis, W3, 64, 64)
    return _make_step_pool(64, 64)(
        sp, t1, t0, dis, W3[4], lam, acc, b3.reshape(1, -1),
        fc1_w, fc1_b.reshape(1, -1), fc2_w, fc2_b.reshape(1, -1))


# speedup vs baseline: 1.3268x; 1.0210x over previous
import functools

import jax
import jax.numpy as jnp
from jax import lax
from jax.experimental import pallas as pl
from jax.experimental.pallas import tpu as pltpu
from jax.experimental.pallas import tpu_sc as plsc

_N = 10000
_E = 320000
_CH = 128
_NCHUNK = _E // _CH
_NC = 2
_NS = 16
_NW = _NC * _NS
_RPT = _N // _NS

_BR = 2000
_NB = _N // _BR

_RPT8 = 624
_TAIL = _N - _NS * _RPT8
_ZCHUNKS = ((0, 128), (128, 128), (256, 128), (384, 128), (512, 112))


def _zero_rows(buf, nrows, ncols):
    def body(i, carry):
        for j in range(ncols // 16):
            buf[i, pl.ds(j * 16, 16)] = jnp.zeros((16,), jnp.float32)
        return carry
    lax.fori_loop(0, nrows, body, 0)


_CPT = _NCHUNK // _NW
_XTRA = _NCHUNK - _CPT * _NW


@functools.lru_cache(maxsize=None)
def _make_sc_scatter(F):
    SR = _CPT
    mesh = plsc.VectorSubcoreMesh(core_axis_name="c", subcore_axis_name="s")

    @functools.partial(
        pl.kernel,
        out_type=jax.ShapeDtypeStruct((_NC, _N, F), jnp.float32),
        mesh=mesh,
        compiler_params=pltpu.CompilerParams(use_tc_tiling_on_sc=False),
        scratch_types=[
            pltpu.VMEM((SR + 1, _CH), jnp.int32),
            pltpu.VMEM((SR + 1, _CH), jnp.int32),
            pltpu.VMEM((4, _CH, F), jnp.float32),
            pltpu.VMEM_SHARED((_N, F), jnp.float32),
            pltpu.SemaphoreType.DMA,
            pltpu.SemaphoreType.DMA,
            pltpu.SemaphoreType.DMA,
            pltpu.SemaphoreType.DMA,
            pltpu.SemaphoreType.DMA,
            pltpu.SemaphoreType.DMA,
            pltpu.SemaphoreType.DMA,
            pltpu.SemaphoreType.DMA,
        ],
    )
    def k(u_hbm, src_hbm, dst_hbm, out_hbm, src_v, dst_v, rows_v, acc_sh,
          g0, g1, g2, g3, s0, s1, s2, s3):
        cid = lax.axis_index("c")
        sid = lax.axis_index("s")
        wid = sid * _NC + cid
        gsem = (g0, g1, g2, g3)
        ssem = (s0, s1, s2, s3)

        _zero_rows(rows_v.at[0], _CH, F)
        zdescs = []
        for off, sz in _ZCHUNKS:
            zdescs.append(pltpu.async_copy(
                rows_v.at[0, pl.ds(0, sz)],
                acc_sh.at[pl.ds(sid * _RPT8 + off, sz)], g0))
        @pl.when(sid == _NS - 1)
        def _():
            pltpu.async_copy(rows_v.at[0, pl.ds(0, _TAIL)],
                             acc_sh.at[pl.ds(_NS * _RPT8, _TAIL)], g1)

        pltpu.sync_copy(src_hbm.at[pl.ds(wid * _CPT, SR)],
                        src_v.at[pl.ds(0, SR)])
        pltpu.sync_copy(dst_hbm.at[pl.ds(wid * _CPT, SR)],
                        dst_v.at[pl.ds(0, SR)])
        @pl.when(wid < _XTRA)
        def _():
            pltpu.sync_copy(src_hbm.at[pl.ds(_NW * _CPT + wid, 1)],
                            src_v.at[pl.ds(SR, 1)])
            pltpu.sync_copy(dst_hbm.at[pl.ds(_NW * _CPT + wid, 1)],
                            dst_v.at[pl.ds(SR, 1)])
        for d in zdescs:
            d.wait()
        @pl.when(sid == _NS - 1)
        def _():
            pltpu.make_async_copy(rows_v.at[0, pl.ds(0, _TAIL)],
                                  acc_sh.at[pl.ds(_NS * _RPT8, _TAIL)],
                                  g1).wait()
        plsc.subcore_barrier()

        def sg(j, b):
            pltpu.async_copy(u_hbm.at[src_v.at[j]], rows_v.at[b], gsem[b])

        def wg(j, b):
            pltpu.make_async_copy(u_hbm.at[src_v.at[j]], rows_v.at[b],
                                  gsem[b]).wait()

        def ss(j, b):
            pltpu.async_copy(rows_v.at[b], acc_sh.at[dst_v.at[j]], ssem[b],
                             add=True)

        def ws(j, b):
            pltpu.make_async_copy(rows_v.at[b], acc_sh.at[dst_v.at[j]],
                                  ssem[b]).wait()

        sg(0, 0)
        sg(1, 1)

        def body(i, carry):
            for b in range(4):
                j = 4 * i + b
                wg(j, b)
                ss(j, b)
                bn = (b + 2) % 4
                if b >= 2:
                    ws(j - 2, bn)
                    sg(j + 2, bn)
                else:
                    @pl.when(i > 0)
                    def _(j=j, bn=bn):
                        ws(j - 2, bn)
                    sg(j + 2, bn)
            return carry

        lax.fori_loop(0, SR // 4, body, 0)
        wg(SR - 2, 0)
        ss(SR - 2, 0)
        ws(SR - 4, 2)
        wg(SR - 1, 1)
        ss(SR - 1, 1)
        ws(SR - 3, 3)
        ws(SR - 2, 0)
        ws(SR - 1, 1)

        @pl.when(wid < _XTRA)
        def _():
            sg(SR, 2)
            wg(SR, 2)
            ss(SR, 2)
            ws(SR, 2)
        plsc.subcore_barrier()

        pltpu.sync_copy(acc_sh.at[pl.ds(sid * _RPT8, _RPT8)],
                        out_hbm.at[cid, pl.ds(sid * _RPT8, _RPT8)])
        @pl.when(sid == _NS - 1)
        def _():
            pltpu.sync_copy(acc_sh.at[pl.ds(_NS * _RPT8, _TAIL)],
                            out_hbm.at[cid, pl.ds(_NS * _RPT8, _TAIL)])

    return k


@functools.lru_cache(maxsize=None)
def _make_sc_degree():
    F = 16
    mesh = plsc.VectorSubcoreMesh(core_axis_name="c", subcore_axis_name="s")

    @functools.partial(
        pl.kernel,
        out_type=jax.ShapeDtypeStruct((_NC, _N, F), jnp.float32),
        mesh=mesh,
        compiler_params=pltpu.CompilerParams(use_tc_tiling_on_sc=False),
        scratch_types=[
            pltpu.VMEM((_CPT + 1, _CH), jnp.int32),
            pltpu.VMEM((_CH, F), jnp.float32),
            pltpu.VMEM_SHARED((_N, F), jnp.float32),
            pltpu.SemaphoreType.DMA,
        ],
    )
    def k(src_hbm, out_hbm, src_v, ones_v, acc_sh, sem):
        cid = lax.axis_index("c")
        sid = lax.axis_index("s")
        wid = sid * _NC + cid

        _zero_rows(ones_v, _CH, F)
        for off, sz in _ZCHUNKS:
            pltpu.sync_copy(ones_v.at[pl.ds(0, sz)],
                            acc_sh.at[pl.ds(sid * _RPT8 + off, sz)])
        @pl.when(sid == _NS - 1)
        def _():
            pltpu.sync_copy(ones_v.at[pl.ds(0, _TAIL)],
                            acc_sh.at[pl.ds(_NS * _RPT8, _TAIL)])

        pltpu.sync_copy(src_hbm.at[pl.ds(wid * _CPT, _CPT)],
                        src_v.at[pl.ds(0, _CPT)])
        @pl.when(wid < _XTRA)
        def _():
            pltpu.sync_copy(src_hbm.at[pl.ds(_NW * _CPT + wid, 1)],
                            src_v.at[pl.ds(_CPT, 1)])
        plsc.subcore_barrier()

        def fill(i, carry):
            ones_v[i, pl.ds(0, 16)] = jnp.ones((16,), jnp.float32)
            return carry
        lax.fori_loop(0, _CH, fill, 0)

        def body(i, carry):
            pltpu.sync_copy(ones_v, acc_sh.at[src_v.at[i]], add=True)
            return carry

        lax.fori_loop(0, _CPT, body, 0)
        @pl.when(wid < _XTRA)
        def _():
            pltpu.sync_copy(ones_v, acc_sh.at[src_v.at[_CPT]], add=True)
        plsc.subcore_barrier()

        pltpu.sync_copy(acc_sh.at[pl.ds(sid * _RPT8, _RPT8)],
                        out_hbm.at[cid, pl.ds(sid * _RPT8, _RPT8)])
        @pl.when(sid == _NS - 1)
        def _():
            pltpu.sync_copy(acc_sh.at[pl.ds(_NS * _RPT8, _TAIL)],
                            out_hbm.at[cid, pl.ds(_NS * _RPT8, _TAIL)])

    return k


def _rb(shape, idx=None):
    if idx is None:
        idx = lambda i: (i,) + (0,) * (len(shape) - 1)
    return pl.BlockSpec(shape, idx)


@functools.lru_cache(maxsize=None)
def _make_prep_clen(Fin, Fo, S):
    def body(degp_ref, x_ref, w_ref, dis_ref, z_ref, u_ref):
        d = degp_ref[0] + degp_ref[1]
        deg = d[:, 0:1]
        dis = jnp.where(deg > 0.0, lax.rsqrt(jnp.maximum(deg, 1e-30)), 0.0)
        dis_ref[...] = dis
        x = x_ref[...]
        for k in range(S):
            zk = jnp.dot(x, w_ref[k], preferred_element_type=jnp.float32)
            z_ref[k] = zk
        u_ref[...] = dis * zk

    return pl.pallas_call(
        body,
        grid=(_NB,),
        in_specs=[
            pl.BlockSpec((_NC, _BR, 16), lambda i: (0, i, 0)),
            _rb((_BR, Fin)),
            pl.BlockSpec((S, Fin, Fo), lambda i: (0, 0, 0)),
        ],
        out_specs=[_rb((_BR, 1)),
                   pl.BlockSpec((S, _BR, Fo), lambda i: (0, i, 0)),
                   _rb((_BR, Fo))],
        out_shape=[
            jax.ShapeDtypeStruct((_N, 1), jnp.float32),
            jax.ShapeDtypeStruct((S, _N, Fo), jnp.float32),
            jax.ShapeDtypeStruct((_N, Fo), jnp.float32),
        ],
    )


@functools.lru_cache(maxsize=None)
def _make_clen_step(zk, bprev_from_z, bpp_kind, Fo):
    def body(*refs):
        it = list(refs)
        sp_ref = it.pop(0)
        bp_ref = it.pop(0)
        bpp_ref = it.pop(0) if bpp_kind != 'none' else None
        z_ref, dis_ref, lam_ref, b_ref, u_ref = it
        s = 2.0 / lam_ref[0]
        dis = dis_ref[...]
        bp = bp_ref[0] if bprev_from_z else bp_ref[...]
        a = (-s) * (dis * (sp_ref[0] + sp_ref[1])) + (s - 1.0) * bp
        b = 2.0 * a + z_ref[0]
        if bpp_ref is not None:
            b = b - (bpp_ref[0] if bpp_kind == 'z' else bpp_ref[...])
        b_ref[...] = b
        u_ref[...] = dis * b

    def zblk(k):
        return pl.BlockSpec((1, _BR, Fo), lambda i, k=k: (k, i, 0))

    in_specs = [pl.BlockSpec((_NC, _BR, Fo), lambda i: (0, i, 0))]
    in_specs.append(zblk(4) if bprev_from_z else _rb((_BR, Fo)))
    if bpp_kind == 'z':
        in_specs.append(zblk(4))
    elif bpp_kind == 'arr':
        in_specs.append(_rb((_BR, Fo)))
    in_specs += [zblk(zk), _rb((_BR, 1)),
                 pl.BlockSpec(memory_space=pltpu.SMEM)]

    return pl.pallas_call(
        body, grid=(_NB,), in_specs=in_specs,
        out_specs=[_rb((_BR, Fo)), _rb((_BR, Fo))],
        out_shape=[jax.ShapeDtypeStruct((_N, Fo), jnp.float32),
                   jax.ShapeDtypeStruct((_N, Fo), jnp.float32)],
    )


@functools.lru_cache(maxsize=None)
def _make_clen_finish_start(Fo, Fo2):
    def body(sp_ref, b1_ref, b2_ref, z_ref, dis_ref, lam_ref, bias_ref,
             w0_ref, h_ref, u_ref, acc_ref):
        s = 2.0 / lam_ref[0]
        dis = dis_ref[...]
        a = (-s) * (dis * (sp_ref[0] + sp_ref[1])) \
            + (s - 1.0) * b1_ref[...]
        out = a - b2_ref[...] + z_ref[0]
        h = jax.nn.relu(out + bias_ref[...])
        h_ref[...] = h
        u_ref[...] = dis * h
        acc_ref[...] = jnp.dot(h, w0_ref[...],
                               preferred_element_type=jnp.float32)

    return pl.pallas_call(
        body,
        grid=(_NB,),
        in_specs=[
            pl.BlockSpec((_NC, _BR, Fo), lambda i: (0, i, 0)),
            _rb((_BR, Fo)),
            _rb((_BR, Fo)),
            pl.BlockSpec((1, _BR, Fo), lambda i: (0, i, 0)),
            _rb((_BR, 1)),
            pl.BlockSpec(memory_space=pltpu.SMEM),
            pl.BlockSpec((1, Fo), lambda i: (0, 0)),
            pl.BlockSpec((Fo, Fo2), lambda i: (0, 0)),
        ],
        out_specs=[_rb((_BR, Fo)), _rb((_BR, Fo)), _rb((_BR, Fo2))],
        out_shape=[
            jax.ShapeDtypeStruct((_N, Fo), jnp.float32),
            jax.ShapeDtypeStruct((_N, Fo), jnp.float32),
            jax.ShapeDtypeStruct((_N, Fo2), jnp.float32),
        ],
    )


@functools.lru_cache(maxsize=None)
def _make_step(m, Fin, Fout, emit_u):
    def body(*refs):
        if m == 2:
            (sp_ref, ta_ref, tb_ref, dis_ref, w_ref, lam_ref, acc_ref,
             *outs) = refs
        else:
            sp_ref, ta_ref, dis_ref, w_ref, lam_ref, acc_ref, *outs = refs
            tb_ref = None
        new_ref, acco_ref = outs[0], outs[1]
        s = 2.0 / lam_ref[0]
        dis = dis_ref[...]
        sm = sp_ref[0] + sp_ref[1]
        new = (-m * s) * (dis * sm) + (m * (s - 1.0)) * ta_ref[...]
        if m == 2:
            new = new - tb_ref[...]
        new_ref[...] = new
        acco_ref[...] = acc_ref[...] + jnp.dot(
            new, w_ref[...], preferred_element_type=jnp.float32)
        if emit_u:
            outs[2][...] = dis * new

    in_specs = [pl.BlockSpec((_NC, _BR, Fin), lambda i: (0, i, 0)),
                _rb((_BR, Fin))]
    if m == 2:
        in_specs.append(_rb((_BR, Fin)))
    in_specs += [
        _rb((_BR, 1)),
        pl.BlockSpec((Fin, Fout), lambda i: (0, 0)),
        pl.BlockSpec(memory_space=pltpu.SMEM),
        _rb((_BR, Fout)),
    ]
    out_specs = [_rb((_BR, Fin)), _rb((_BR, Fout))]
    out_shape = [jax.ShapeDtypeStruct((_N, Fin), jnp.float32),
                 jax.ShapeDtypeStruct((_N, Fout), jnp.float32)]
    if emit_u:
        out_specs.append(_rb((_BR, Fin)))
        out_shape.append(jax.ShapeDtypeStruct((_N, Fin), jnp.float32))

    return pl.pallas_call(
        body, grid=(_NB,), in_specs=in_specs,
        out_specs=out_specs, out_shape=out_shape,
    )


@functools.lru_cache(maxsize=None)
def _make_step_start(Fin, Fout, Fo2):
    def body(sp_ref, ta_ref, tb_ref, dis_ref, w4_ref, w0_ref, lam_ref,
             acc_ref, b_ref, h_ref, u_ref, acco_ref):
        s = 2.0 / lam_ref[0]
        dis = dis_ref[...]
        sm = sp_ref[0] + sp_ref[1]
        new = (-2.0 * s) * (dis * sm) + (2.0 * (s - 1.0)) * ta_ref[...] \
            - tb_ref[...]
        acc4 = acc_ref[...] + jnp.dot(new, w4_ref[...],
                                      preferred_element_type=jnp.float32)
        h = jax.nn.relu(acc4 + b_ref[...])
        h_ref[...] = h
        u_ref[...] = dis * h
        acco_ref[...] = jnp.dot(h, w0_ref[...],
                                preferred_element_type=jnp.float32)

    return pl.pallas_call(
        body,
        grid=(_NB,),
        in_specs=[
            pl.BlockSpec((_NC, _BR, Fin), lambda i: (0, i, 0)),
            _rb((_BR, Fin)),
            _rb((_BR, Fin)),
            _rb((_BR, 1)),
            pl.BlockSpec((Fin, Fout), lambda i: (0, 0)),
            pl.BlockSpec((Fout, Fo2), lambda i: (0, 0)),
            pl.BlockSpec(memory_space=pltpu.SMEM),
            _rb((_BR, Fout)),
            pl.BlockSpec((1, Fout), lambda i: (0, 0)),
        ],
        out_specs=[_rb((_BR, Fout)), _rb((_BR, Fout)), _rb((_BR, Fo2))],
        out_shape=[
            jax.ShapeDtypeStruct((_N, Fout), jnp.float32),
            jax.ShapeDtypeStruct((_N, Fout), jnp.float32),
            jax.ShapeDtypeStruct((_N, Fo2), jnp.float32),
        ],
    )


@functools.lru_cache(maxsize=None)
def _make_step_pool(Fin, Fout):
    def body(sp_ref, ta_ref, tb_ref, dis_ref, w4_ref, lam_ref, acc_ref,
             b_ref, w1_ref, b1_ref, w2_ref, b2_ref, out_ref, sum_ref):
        i = pl.program_id(0)
        s = 2.0 / lam_ref[0]
        dis = dis_ref[...]
        sm = sp_ref[0] + sp_ref[1]
        new = (-2.0 * s) * (dis * sm) + (2.0 * (s - 1.0)) * ta_ref[...] \
            - tb_ref[...]
        acc4 = acc_ref[...] + jnp.dot(new, w4_ref[...],
                                      preferred_element_type=jnp.float32)
        h = jax.nn.relu(acc4 + b_ref[...])
        part = jnp.sum(h, axis=0, keepdims=True)

        @pl.when(i == 0)
        def _():
            sum_ref[...] = part

        @pl.when(i > 0)
        def _():
            sum_ref[...] = sum_ref[...] + part

        @pl.when(i == _NB - 1)
        def _():
            pooled = sum_ref[...] / jnp.float32(_N)
            t = jax.nn.relu(
                jnp.dot(pooled, w1_ref[...],
                        preferred_element_type=jnp.float32) + b1_ref[...])
            out_ref[...] = jnp.dot(
                t, w2_ref[...], preferred_element_type=jnp.float32) \
                + b2_ref[...]

    return pl.pallas_call(
        body,
        grid=(_NB,),
        in_specs=[
            pl.BlockSpec((_NC, _BR, Fin), lambda i: (0, i, 0)),
            _rb((_BR, Fin)),
            _rb((_BR, Fin)),
            _rb((_BR, 1)),
            pl.BlockSpec((Fin, Fout), lambda i: (0, 0)),
            pl.BlockSpec(memory_space=pltpu.SMEM),
            _rb((_BR, Fout)),
            pl.BlockSpec((1, Fout), lambda i: (0, 0)),
            pl.BlockSpec((Fout, 10), lambda i: (0, 0)),
            pl.BlockSpec((1, 10), lambda i: (0, 0)),
            pl.BlockSpec((10, 10), lambda i: (0, 0)),
            pl.BlockSpec((1, 10), lambda i: (0, 0)),
        ],
        out_specs=pl.BlockSpec((1, 10), lambda i: (0, 0)),
        out_shape=jax.ShapeDtypeStruct((1, 10), jnp.float32),
        scratch_shapes=[pltpu.VMEM((1, Fout), jnp.float32)],
    )


def kernel(x, edge_index, lambda_max, batch, W1, b1, W2, b2, W3, b3,
           fc1_w, fc1_b, fc2_w, fc2_b):
    src = edge_index[0].reshape(_NCHUNK, _CH)
    dst = edge_index[1].reshape(_NCHUNK, _CH)
    lam = lambda_max.reshape((1,)).astype(jnp.float32)

    degp = _make_sc_degree()(src)

    dis, z, u = _make_prep_clen(128, 32, 5)(degp, x, W1)
    scat32 = _make_sc_scatter(32)
    sp = scat32(u, src, dst)
    cb3, u = _make_clen_step(3, True, 'none', 32)(sp, z, z, dis, lam)
    sp = scat32(u, src, dst)
    cb2, u = _make_clen_step(2, False, 'z', 32)(sp, cb3, z, z, dis, lam)
    sp = scat32(u, src, dst)
    cb1, u = _make_clen_step(1, False, 'arr', 32)(sp, cb2, cb3, z, dis, lam)
    sp = scat32(u, src, dst)
    h, u, acc = _make_clen_finish_start(32, 64)(
        sp, cb1, cb2, z, dis, lam, b1.reshape(1, -1), W2[0])

    def run_3hops(u, acc, t0, dis, W, Fin, Fout):
        scat = _make_sc_scatter(Fin)
        sp = scat(u, src, dst)
        t1, acc, u = _make_step(1, Fin, Fout, True)(sp, t0, dis, W[1], lam,
                                                    acc)
        for k in (2, 3):
            sp = scat(u, src, dst)
            t2, acc, u = _make_step(2, Fin, Fout, True)(sp, t1, t0, dis,
                                                        W[k], lam, acc)
            t0, t1 = t1, t2
        sp = scat(u, src, dst)
        return sp, t1, t0, acc

    sp, t1, t0, acc = run_3hops(u, acc, h, dis, W2, 32, 64)
    h, u, acc = _make_step_start(32, 64, 64)(
        sp, t1, t0, dis, W2[4], W3[0], lam, acc, b2.reshape(1, -1))

    sp, t1, t0, acc = run_3hops(u, acc, h, dis, W3, 64, 64)
    return _make_step_pool(64, 64)(
        sp, t1, t0, dis, W3[4], lam, acc, b3.reshape(1, -1),
        fc1_w, fc1_b.reshape(1, -1), fc2_w, fc2_b.reshape(1, -1))
